# initial kernel scaffold (unmeasured)
import jax
import jax.numpy as jnp
from jax import lax
from jax.experimental import pallas as pl
from jax.experimental.pallas import tpu as pltpu


def kernel(
    x,
):
    def body(*refs):
        pass

    out_shape = jax.ShapeDtypeStruct(..., jnp.float32)
    return pl.pallas_call(body, out_shape=out_shape)(...)


# baseline (device time: 12042 ns/iter reference)
import jax
import jax.numpy as jnp
from jax import lax
from jax.experimental import pallas as pl
from jax.experimental.pallas import tpu as pltpu


def kernel(x):
    m, n2 = x.shape
    n = n2 // 2
    M = 2 * m

    def body(x_ref, out_ref, send_sem, recv_sem):
        mx = lax.axis_index("x")
        my = lax.axis_index("y")

        rdma = pltpu.make_async_remote_copy(
            src_ref=x_ref.at[:, pl.ds((1 - mx) * n, n)],
            dst_ref=out_ref.at[pl.ds(mx * m, m), :],
            send_sem=send_sem,
            recv_sem=recv_sem,
            device_id=(1 - mx, my),
            device_id_type=pl.DeviceIdType.MESH,
        )
        rdma.start()

        out_ref[pl.ds(mx * m, m), :] = x_ref[:, pl.ds(mx * n, n)]

        rdma.wait()

    out_shape = jax.ShapeDtypeStruct((M, n), x.dtype)
    return pl.pallas_call(
        body,
        out_shape=out_shape,
        in_specs=[pl.BlockSpec(memory_space=pltpu.VMEM)],
        out_specs=pl.BlockSpec(memory_space=pltpu.VMEM),
        scratch_shapes=[
            pltpu.SemaphoreType.DMA,
            pltpu.SemaphoreType.DMA,
        ],
    )(x)


# device time: 8247 ns/iter; 1.4602x vs baseline; 1.4602x over previous
import jax
import jax.numpy as jnp
from jax import lax
from jax.experimental import pallas as pl
from jax.experimental.pallas import tpu as pltpu


def kernel(x):
    m, n2 = x.shape
    n = n2 // 2
    M = 2 * m

    def body(x_ref, out_ref, send_sem, recv_sem):
        mx = lax.axis_index("x")
        my = lax.axis_index("y")

        barrier_sem = pltpu.get_barrier_semaphore()
        pl.semaphore_signal(
            barrier_sem,
            inc=1,
            device_id=(1 - mx, my),
            device_id_type=pl.DeviceIdType.MESH,
        )
        pl.semaphore_wait(barrier_sem, 1)

        rdma = pltpu.make_async_remote_copy(
            src_ref=x_ref.at[:, pl.ds((1 - mx) * n, n)],
            dst_ref=out_ref.at[pl.ds(mx * m, m), :],
            send_sem=send_sem,
            recv_sem=recv_sem,
            device_id=(1 - mx, my),
            device_id_type=pl.DeviceIdType.MESH,
        )
        rdma.start()

        out_ref[pl.ds(mx * m, m), :] = x_ref[:, pl.ds(mx * n, n)]

        rdma.wait()

    out_shape = jax.ShapeDtypeStruct((M, n), x.dtype)
    return pl.pallas_call(
        body,
        out_shape=out_shape,
        in_specs=[pl.BlockSpec(memory_space=pltpu.VMEM)],
        out_specs=pl.BlockSpec(memory_space=pltpu.VMEM),
        scratch_shapes=[
            pltpu.SemaphoreType.DMA,
            pltpu.SemaphoreType.DMA,
        ],
        compiler_params=pltpu.CompilerParams(collective_id=0),
    )(x)
